# Initial kernel scaffold; baseline (speedup 1.0000x reference)
import jax
import jax.numpy as jnp
from jax.experimental import pallas as pl


def kernel(X, A, W1, b1, W2, b2, W3, b3):
    raise NotImplementedError("write your pallas kernel here")


# speedup vs baseline: 10.5503x; 10.5503x over previous
import functools

import jax
import jax.numpy as jnp
from jax import lax
from jax.experimental import pallas as pl
from jax.experimental.pallas import tpu as pltpu
from jax.experimental.pallas import tpu_sc as plsc

F = 16
FH = 8
NC = 2
NS = 16
CH = 128
BLK = 1024
_RS2 = 0.7071067811865476


def _round_up(x, m):
    return (x + m - 1) // m * m


def _make_deg_kernel(Np, E_pad):
    W = NC * NS
    EP = E_pad // W
    NCH = EP // CH
    RPT = Np // NS
    mesh = plsc.VectorSubcoreMesh(core_axis_name="c", subcore_axis_name="s",
                                  num_cores=NC)

    @functools.partial(
        pl.kernel,
        mesh=mesh,
        compiler_params=pltpu.CompilerParams(use_tc_tiling_on_sc=False),
        out_type=jax.ShapeDtypeStruct((NC * Np,), jnp.float32),
        scratch_types=[
            pltpu.VMEM((CH,), jnp.int32),
            pltpu.VMEM((CH,), jnp.float32),
            pltpu.VMEM((RPT,), jnp.float32),
            pltpu.VMEM_SHARED((Np,), jnp.float32),
        ],
    )
    def degk(dst_hbm, out_hbm, idx_d, ones_v, stage, acc):
        cid = lax.axis_index("c")
        sid = lax.axis_index("s")

        def zfill(j, carry):
            stage[pl.ds(j * 16, 16)] = jnp.zeros((16,), jnp.float32)
            return carry
        lax.fori_loop(0, RPT // 16, zfill, 0)
        pltpu.sync_copy(stage, acc.at[pl.ds(sid * RPT, RPT)])
        for i in range(CH // 16):
            ones_v[pl.ds(i * 16, 16)] = jnp.full((16,), 1.0, jnp.float32)
        plsc.subcore_barrier()
        base = (cid * NS + sid) * EP

        def body(j, carry):
            off = base + j * CH
            pltpu.sync_copy(dst_hbm.at[pl.ds(off, CH)], idx_d)
            pltpu.sync_copy(ones_v, acc.at[idx_d], add=True)
            return carry

        lax.fori_loop(0, NCH, body, 0)
        plsc.subcore_barrier()
        pltpu.sync_copy(acc.at[pl.ds(sid * RPT, RPT)], stage)
        pltpu.sync_copy(stage, out_hbm.at[pl.ds(cid * Np + sid * RPT, RPT)])

    return degk


def _make_prop_kernel(Np, E_pad):
    EP = E_pad // NS
    NCH = EP // CH
    RPT = Np // NS
    ZB = RPT // 8
    mesh = plsc.VectorSubcoreMesh(core_axis_name="c", subcore_axis_name="s",
                                  num_cores=NC)

    @functools.partial(
        pl.kernel,
        mesh=mesh,
        compiler_params=pltpu.CompilerParams(use_tc_tiling_on_sc=False),
        out_type=jax.ShapeDtypeStruct((NC * Np, FH), jnp.float32),
        scratch_types=[
            pltpu.VMEM((CH,), jnp.int32),
            pltpu.VMEM((CH,), jnp.int32),
            pltpu.VMEM((CH, FH), jnp.float32),
            pltpu.VMEM((ZB, FH), jnp.float32),
            pltpu.VMEM_SHARED((Np, FH), jnp.float32),
            pltpu.SemaphoreType.DMA,
        ],
    )
    def prop(src_hbm, dst_hbm, tlo_hbm, thi_hbm, out_hbm,
             idx_s, idx_d, rows, stage, acc, sem):
        cid = lax.axis_index("c")
        sid = lax.axis_index("s")

        def zfill(j, carry):
            stage[j, :] = jnp.zeros((FH,), jnp.float32)
            return carry
        lax.fori_loop(0, ZB, zfill, 0)

        def zcopy(k, carry):
            pltpu.sync_copy(stage, acc.at[pl.ds(sid * RPT + k * ZB, ZB)])
            return carry
        lax.fori_loop(0, 8, zcopy, 0)
        plsc.subcore_barrier()
        base = sid * EP

        def run_edges(table):
            def body(j, carry):
                off = base + j * CH
                pltpu.sync_copy(src_hbm.at[pl.ds(off, CH)], idx_s)
                pltpu.sync_copy(dst_hbm.at[pl.ds(off, CH)], idx_d)
                pltpu.async_copy(table.at[idx_s], rows, sem).wait()
                pltpu.sync_copy(rows, acc.at[idx_d], add=True)
                return carry
            lax.fori_loop(0, NCH, body, 0)

        @pl.when(cid == 0)
        def _():
            run_edges(tlo_hbm)

        @pl.when(cid == 1)
        def _():
            run_edges(thi_hbm)

        plsc.subcore_barrier()

        def ocopy(k, carry):
            pltpu.sync_copy(acc.at[pl.ds(sid * RPT + k * ZB, ZB)], stage)
            pltpu.sync_copy(
                stage, out_hbm.at[pl.ds(cid * Np + sid * RPT + k * ZB, ZB)])
            return carry
        lax.fori_loop(0, 8, ocopy, 0)

    return prop


def _row_grid(Np):
    return (pl.cdiv(Np, BLK),)


def _bspecF():
    return pl.BlockSpec((BLK, F), lambda i: (i, 0))


def _bspecH():
    return pl.BlockSpec((BLK, FH), lambda i: (i, 0))


def _bspec1():
    return pl.BlockSpec((BLK, 1), lambda i: (i, 0))


def _bspecW():
    return pl.BlockSpec((F, F), lambda i: (0, 0))


def _bspecB():
    return pl.BlockSpec((1, F), lambda i: (0, 0))


def _dis_body(d0, d1, o):
    o[...] = lax.rsqrt(d0[...] + d1[...] + 1.0)


def _make_dis_kernel(Np):
    return pl.pallas_call(
        _dis_body,
        grid=_row_grid(Np),
        in_specs=[_bspec1(), _bspec1()],
        out_specs=_bspec1(),
        out_shape=jax.ShapeDtypeStruct((Np, 1), jnp.float32),
    )


def _in_body(x, dis, w, olo, ohi):
    r = dis[...] * jnp.dot(x[...], w[...], preferred_element_type=jnp.float32)
    olo[...] = r[:, :FH]
    ohi[...] = r[:, FH:]


def _make_in_kernel(Np):
    return pl.pallas_call(
        _in_body,
        grid=_row_grid(Np),
        in_specs=[_bspecF(), _bspec1(), _bspecW()],
        out_specs=[_bspecH(), _bspecH()],
        out_shape=[jax.ShapeDtypeStruct((Np, FH), jnp.float32),
                   jax.ShapeDtypeStruct((Np, FH), jnp.float32)],
    )


def _mid_body(plo, phi, xlo, xhi, dis, b, w, olo, ohi):
    d = dis[...]
    p = jnp.concatenate([plo[...], phi[...]], axis=1)
    xt = jnp.concatenate([xlo[...], xhi[...]], axis=1)
    z = d * (p + xt) + b[...]
    h = 0.5 * z * (1.0 + lax.erf(z * _RS2))
    r = d * jnp.dot(h, w[...], preferred_element_type=jnp.float32)
    olo[...] = r[:, :FH]
    ohi[...] = r[:, FH:]


def _make_mid_kernel(Np):
    return pl.pallas_call(
        _mid_body,
        grid=_row_grid(Np),
        in_specs=[_bspecH(), _bspecH(), _bspecH(), _bspecH(),
                  _bspec1(), _bspecB(), _bspecW()],
        out_specs=[_bspecH(), _bspecH()],
        out_shape=[jax.ShapeDtypeStruct((Np, FH), jnp.float32),
                   jax.ShapeDtypeStruct((Np, FH), jnp.float32)],
    )


def _make_out_kernel(Np, OUT):
    def body(plo, phi, xlo, xhi, dis, b, o):
        p = jnp.concatenate([plo[...], phi[...]], axis=1)
        xt = jnp.concatenate([xlo[...], xhi[...]], axis=1)
        z = dis[...] * (p + xt) + b[...]
        col = lax.broadcasted_iota(jnp.int32, (BLK, F), 1)
        valid = col < OUT
        zm = jnp.where(valid, z, -jnp.inf)
        m = jnp.max(zm, axis=1, keepdims=True)
        e = jnp.where(valid, jnp.exp(z - m), 0.0)
        s = jnp.sum(e, axis=1, keepdims=True)
        o[...] = (z - m - jnp.log(s))[:, :OUT]

    return pl.pallas_call(
        body,
        grid=_row_grid(Np),
        in_specs=[_bspecH(), _bspecH(), _bspecH(), _bspecH(),
                  _bspec1(), _bspecB()],
        out_specs=pl.BlockSpec((BLK, OUT), lambda i: (i, 0)),
        out_shape=jax.ShapeDtypeStruct((Np, OUT), jnp.float32),
    )


def kernel(X, A, W1, b1, W2, b2, W3, b3):
    N, K = X.shape
    E = A.shape[1]
    OUT = W3.shape[1]
    Np = _round_up(N, 256) + 256
    E_pad = _round_up(E, NC * NS * CH)

    src = A[0].astype(jnp.int32)
    dst = A[1].astype(jnp.int32)
    npad = E_pad - E
    pad_node = N + (jnp.arange(npad, dtype=jnp.int32) % 128)
    srcp = jnp.concatenate([src, pad_node])
    dstp = jnp.concatenate([dst, pad_node])

    X_pad = jnp.pad(X, ((0, Np - N), (0, F - K)))
    W1p = jnp.pad(W1, ((0, F - K), (0, F - K)))
    W2p = jnp.pad(W2, ((0, F - K), (0, F - K)))
    W3p = jnp.pad(W3, ((0, F - K), (0, F - OUT)))
    b1p = jnp.pad(b1, (0, F - K)).reshape(1, F)
    b2p = jnp.pad(b2, (0, F - K)).reshape(1, F)
    b3p = jnp.pad(b3, (0, F - OUT)).reshape(1, F)

    degk = _make_deg_kernel(Np, E_pad)
    prop = _make_prop_kernel(Np, E_pad)
    k_dis = _make_dis_kernel(Np)
    k_in = _make_in_kernel(Np)
    k_mid = _make_mid_kernel(Np)
    k_out = _make_out_kernel(Np, OUT)

    degp = degk(dstp)
    dis = k_dis(degp[:Np].reshape(Np, 1), degp[Np:].reshape(Np, 1))

    lo, hi = k_in(X_pad, dis, W1p)
    p = prop(srcp, dstp, lo, hi)
    lo, hi = k_mid(p[:Np], p[Np:], lo, hi, dis, b1p, W2p)
    p = prop(srcp, dstp, lo, hi)
    lo, hi = k_mid(p[:Np], p[Np:], lo, hi, dis, b2p, W3p)
    p = prop(srcp, dstp, lo, hi)
    out = k_out(p[:Np], p[Np:], lo, hi, dis, b3p)
    return out[:N]


# speedup vs baseline: 40.1219x; 3.8029x over previous
import functools

import jax
import jax.numpy as jnp
from jax import lax
from jax.experimental import pallas as pl
from jax.experimental.pallas import tpu as pltpu
from jax.experimental.pallas import tpu_sc as plsc

F = 16
FH = 8
NC = 2
NS = 16
CH = 128
G = 4
KI = 16
BLK = 1024
_RS2 = 0.7071067811865476


def _round_up(x, m):
    return (x + m - 1) // m * m


def _make_deg_kernel(Np, E_pad):
    W = NC * NS
    EP = E_pad // W
    RPW = EP // CH
    NB = RPW // KI
    RPT = Np // NS
    mesh = plsc.VectorSubcoreMesh(core_axis_name="c", subcore_axis_name="s",
                                  num_cores=NC)

    @functools.partial(
        pl.kernel,
        mesh=mesh,
        compiler_params=pltpu.CompilerParams(use_tc_tiling_on_sc=False),
        out_type=jax.ShapeDtypeStruct((NC * Np,), jnp.float32),
        scratch_types=[
            pltpu.VMEM((2 * KI, CH), jnp.int32),
            pltpu.VMEM((CH,), jnp.float32),
            pltpu.VMEM((RPT,), jnp.float32),
            pltpu.VMEM_SHARED((Np,), jnp.float32),
            pltpu.SemaphoreType.DMA,
            pltpu.SemaphoreType.DMA,
        ],
    )
    def degk(dst2_hbm, out_hbm, idxd, ones_v, stage, acc, isem, ssem):
        cid = lax.axis_index("c")
        sid = lax.axis_index("s")

        def zfill(j, carry):
            stage[pl.ds(j * 16, 16)] = jnp.zeros((16,), jnp.float32)
            return carry
        lax.fori_loop(0, RPT // 16, zfill, 0)
        pltpu.sync_copy(stage, acc.at[pl.ds(sid * RPT, RPT)])
        for i in range(CH // 16):
            ones_v[pl.ds(i * 16, 16)] = jnp.full((16,), 1.0, jnp.float32)
        plsc.subcore_barrier()

        row0 = (cid * NS + sid) * RPW

        def idx_cp(block, slot):
            return pltpu.make_async_copy(
                dst2_hbm.at[pl.ds(row0 + block * KI, KI)],
                idxd.at[pl.ds(slot * KI, KI)], isem)

        idx_cp(0, 0).start()

        @pl.loop(0, NB, step=2)
        def _blocks(kv):
            for p in (0, 1):
                k = kv + p
                idx_cp(k, p).wait()

                @pl.when(k + 1 < NB)
                def _():
                    idx_cp(k + 1, p ^ 1).start()

                for hb in range(2):
                    sc = [pltpu.make_async_copy(
                        ones_v, acc.at[idxd.at[p * KI + hb * (KI // 2) + r]],
                        ssem) for r in range(KI // 2)]
                    for c in sc:
                        c.start(add=True)
                    for c in sc:
                        c.wait()

        plsc.subcore_barrier()
        pltpu.sync_copy(acc.at[pl.ds(sid * RPT, RPT)], stage)
        pltpu.sync_copy(stage, out_hbm.at[pl.ds(cid * Np + sid * RPT, RPT)])

    return degk


def _make_prop_kernel(Np, E_pad):
    EP = E_pad // NS
    RPW = EP // CH
    NB = RPW // KI
    RPT = Np // NS
    ZB = RPT // 8
    mesh = plsc.VectorSubcoreMesh(core_axis_name="c", subcore_axis_name="s",
                                  num_cores=NC)

    @functools.partial(
        pl.kernel,
        mesh=mesh,
        compiler_params=pltpu.CompilerParams(use_tc_tiling_on_sc=False),
        out_type=jax.ShapeDtypeStruct((NC * Np, FH), jnp.float32),
        scratch_types=[
            pltpu.VMEM((2 * KI, CH), jnp.int32),
            pltpu.VMEM((2 * KI, CH), jnp.int32),
            pltpu.VMEM((2 * G, CH, FH), jnp.float32),
            pltpu.VMEM((ZB, FH), jnp.float32),
            pltpu.VMEM_SHARED((Np, FH), jnp.float32),
            pltpu.SemaphoreType.DMA,
            pltpu.SemaphoreType.DMA,
            pltpu.SemaphoreType.DMA,
        ],
    )
    def prop(src2_hbm, dst2_hbm, tlo_hbm, thi_hbm, out_hbm,
             idxs, idxd, rows, stage, acc, isem, gsem, ssem):
        cid = lax.axis_index("c")
        sid = lax.axis_index("s")

        def zfill(j, carry):
            stage[j, :] = jnp.zeros((FH,), jnp.float32)
            return carry
        lax.fori_loop(0, ZB, zfill, 0)

        def zcopy(k, carry):
            pltpu.sync_copy(stage, acc.at[pl.ds(sid * RPT + k * ZB, ZB)])
            return carry
        lax.fori_loop(0, 8, zcopy, 0)
        plsc.subcore_barrier()

        row0 = sid * RPW

        def idx_cp(ref, buf, block, slot):
            return pltpu.make_async_copy(
                ref.at[pl.ds(row0 + block * KI, KI)],
                buf.at[pl.ds(slot * KI, KI)], isem)

        def gat_cp(table, p, h, i):
            return pltpu.make_async_copy(
                table.at[idxs.at[p * KI + h * G + i]],
                rows.at[(h % 2) * G + i], gsem)

        def sca_cp(p, h, i):
            return pltpu.make_async_copy(
                rows.at[(h % 2) * G + i],
                acc.at[idxd.at[p * KI + h * G + i]], ssem)

        def run_edges(table):
            idx_cp(src2_hbm, idxs, 0, 0).start()
            idx_cp(dst2_hbm, idxd, 0, 0).start()

            @pl.loop(0, NB, step=2)
            def _blocks(kv):
                for p in (0, 1):
                    k = kv + p
                    idx_cp(src2_hbm, idxs, k, p).wait()
                    idx_cp(dst2_hbm, idxd, k, p).wait()

                    @pl.when(k + 1 < NB)
                    def _():
                        idx_cp(src2_hbm, idxs, k + 1, p ^ 1).start()
                        idx_cp(dst2_hbm, idxd, k + 1, p ^ 1).start()

                    for h in range(KI // G):
                        gs = [gat_cp(table, p, h, i) for i in range(G)]
                        for c in gs:
                            c.start()
                        for c in gs:
                            c.wait()
                        sc = [sca_cp(p, h, i) for i in range(G)]
                        for c in sc:
                            c.start(add=True)
                        for c in sc:
                            c.wait()

        @pl.when(cid == 0)
        def _():
            run_edges(tlo_hbm)

        @pl.when(cid == 1)
        def _():
            run_edges(thi_hbm)

        plsc.subcore_barrier()

        def ocopy(k, carry):
            pltpu.sync_copy(acc.at[pl.ds(sid * RPT + k * ZB, ZB)], stage)
            pltpu.sync_copy(
                stage, out_hbm.at[pl.ds(cid * Np + sid * RPT + k * ZB, ZB)])
            return carry
        lax.fori_loop(0, 8, ocopy, 0)

    return prop


def _row_grid(Np):
    return (pl.cdiv(Np, BLK),)


def _bspecF():
    return pl.BlockSpec((BLK, F), lambda i: (i, 0))


def _bspecH():
    return pl.BlockSpec((BLK, FH), lambda i: (i, 0))


def _bspec1():
    return pl.BlockSpec((BLK, 1), lambda i: (i, 0))


def _bspecW():
    return pl.BlockSpec((F, F), lambda i: (0, 0))


def _bspecB():
    return pl.BlockSpec((1, F), lambda i: (0, 0))


def _dis_body(d0, d1, o):
    o[...] = lax.rsqrt(d0[...] + d1[...] + 1.0)


def _make_dis_kernel(Np):
    return pl.pallas_call(
        _dis_body,
        grid=_row_grid(Np),
        in_specs=[_bspec1(), _bspec1()],
        out_specs=_bspec1(),
        out_shape=jax.ShapeDtypeStruct((Np, 1), jnp.float32),
    )


def _in_body(x, dis, w, olo, ohi):
    r = dis[...] * jnp.dot(x[...], w[...], preferred_element_type=jnp.float32)
    olo[...] = r[:, :FH]
    ohi[...] = r[:, FH:]


def _make_in_kernel(Np):
    return pl.pallas_call(
        _in_body,
        grid=_row_grid(Np),
        in_specs=[_bspecF(), _bspec1(), _bspecW()],
        out_specs=[_bspecH(), _bspecH()],
        out_shape=[jax.ShapeDtypeStruct((Np, FH), jnp.float32),
                   jax.ShapeDtypeStruct((Np, FH), jnp.float32)],
    )


def _mid_body(plo, phi, xlo, xhi, dis, b, w, olo, ohi):
    d = dis[...]
    p = jnp.concatenate([plo[...], phi[...]], axis=1)
    xt = jnp.concatenate([xlo[...], xhi[...]], axis=1)
    z = d * (p + xt) + b[...]
    h = 0.5 * z * (1.0 + lax.erf(z * _RS2))
    r = d * jnp.dot(h, w[...], preferred_element_type=jnp.float32)
    olo[...] = r[:, :FH]
    ohi[...] = r[:, FH:]


def _make_mid_kernel(Np):
    return pl.pallas_call(
        _mid_body,
        grid=_row_grid(Np),
        in_specs=[_bspecH(), _bspecH(), _bspecH(), _bspecH(),
                  _bspec1(), _bspecB(), _bspecW()],
        out_specs=[_bspecH(), _bspecH()],
        out_shape=[jax.ShapeDtypeStruct((Np, FH), jnp.float32),
                   jax.ShapeDtypeStruct((Np, FH), jnp.float32)],
    )


def _make_out_kernel(Np, OUT):
    def body(plo, phi, xlo, xhi, dis, b, o):
        p = jnp.concatenate([plo[...], phi[...]], axis=1)
        xt = jnp.concatenate([xlo[...], xhi[...]], axis=1)
        z = dis[...] * (p + xt) + b[...]
        col = lax.broadcasted_iota(jnp.int32, (BLK, F), 1)
        valid = col < OUT
        zm = jnp.where(valid, z, -jnp.inf)
        m = jnp.max(zm, axis=1, keepdims=True)
        e = jnp.where(valid, jnp.exp(z - m), 0.0)
        s = jnp.sum(e, axis=1, keepdims=True)
        o[...] = (z - m - jnp.log(s))[:, :OUT]

    return pl.pallas_call(
        body,
        grid=_row_grid(Np),
        in_specs=[_bspecH(), _bspecH(), _bspecH(), _bspecH(),
                  _bspec1(), _bspecB()],
        out_specs=pl.BlockSpec((BLK, OUT), lambda i: (i, 0)),
        out_shape=jax.ShapeDtypeStruct((Np, OUT), jnp.float32),
    )


def kernel(X, A, W1, b1, W2, b2, W3, b3):
    N, K = X.shape
    E = A.shape[1]
    OUT = W3.shape[1]
    Np = _round_up(N, 256) + 256
    E_pad = _round_up(E, NC * NS * CH * KI)

    src = A[0].astype(jnp.int32)
    dst = A[1].astype(jnp.int32)
    npad = E_pad - E
    pad_node = N + (jnp.arange(npad, dtype=jnp.int32) % 128)
    srcp = jnp.concatenate([src, pad_node]).reshape(E_pad // CH, CH)
    dstp = jnp.concatenate([dst, pad_node]).reshape(E_pad // CH, CH)

    X_pad = jnp.pad(X, ((0, Np - N), (0, F - K)))
    W1p = jnp.pad(W1, ((0, F - K), (0, F - K)))
    W2p = jnp.pad(W2, ((0, F - K), (0, F - K)))
    W3p = jnp.pad(W3, ((0, F - K), (0, F - OUT)))
    b1p = jnp.pad(b1, (0, F - K)).reshape(1, F)
    b2p = jnp.pad(b2, (0, F - K)).reshape(1, F)
    b3p = jnp.pad(b3, (0, F - OUT)).reshape(1, F)

    degk = _make_deg_kernel(Np, E_pad)
    prop = _make_prop_kernel(Np, E_pad)
    k_dis = _make_dis_kernel(Np)
    k_in = _make_in_kernel(Np)
    k_mid = _make_mid_kernel(Np)
    k_out = _make_out_kernel(Np, OUT)

    degp = degk(dstp)
    dis = k_dis(degp[:Np].reshape(Np, 1), degp[Np:].reshape(Np, 1))

    lo, hi = k_in(X_pad, dis, W1p)
    p = prop(srcp, dstp, lo, hi)
    lo, hi = k_mid(p[:Np], p[Np:], lo, hi, dis, b1p, W2p)
    p = prop(srcp, dstp, lo, hi)
    lo, hi = k_mid(p[:Np], p[Np:], lo, hi, dis, b2p, W3p)
    p = prop(srcp, dstp, lo, hi)
    out = k_out(p[:Np], p[Np:], lo, hi, dis, b3p)
    return out[:N]


# speedup vs baseline: 43.9636x; 1.0958x over previous
import functools

import jax
import jax.numpy as jnp
from jax import lax
from jax.experimental import pallas as pl
from jax.experimental.pallas import tpu as pltpu
from jax.experimental.pallas import tpu_sc as plsc

F = 16
FH = 8
NC = 2
NS = 16
CH = 128
G = 4
KI = 16
BLK = 1024
_RS2 = 0.7071067811865476


def _round_up(x, m):
    return (x + m - 1) // m * m


def _make_deg_kernel(Np, E_pad):
    W = NC * NS
    EP = E_pad // W
    RPW = EP // CH
    NB = RPW // KI
    RPT = Np // NS
    mesh = plsc.VectorSubcoreMesh(core_axis_name="c", subcore_axis_name="s",
                                  num_cores=NC)

    @functools.partial(
        pl.kernel,
        mesh=mesh,
        compiler_params=pltpu.CompilerParams(use_tc_tiling_on_sc=False),
        out_type=jax.ShapeDtypeStruct((NC * Np,), jnp.float32),
        scratch_types=[
            pltpu.VMEM((2 * KI, CH), jnp.int32),
            pltpu.VMEM((CH,), jnp.float32),
            pltpu.VMEM((RPT,), jnp.float32),
            pltpu.VMEM_SHARED((Np,), jnp.float32),
            pltpu.SemaphoreType.DMA,
            pltpu.SemaphoreType.DMA,
        ],
    )
    def degk(dst2_hbm, out_hbm, idxd, ones_v, stage, acc, isem, ssem):
        cid = lax.axis_index("c")
        sid = lax.axis_index("s")

        def zfill(j, carry):
            stage[pl.ds(j * 16, 16)] = jnp.zeros((16,), jnp.float32)
            return carry
        lax.fori_loop(0, RPT // 16, zfill, 0)
        pltpu.sync_copy(stage, acc.at[pl.ds(sid * RPT, RPT)])
        for i in range(CH // 16):
            ones_v[pl.ds(i * 16, 16)] = jnp.full((16,), 1.0, jnp.float32)
        plsc.subcore_barrier()

        row0 = (cid * NS + sid) * RPW

        def idx_cp(block, slot):
            return pltpu.make_async_copy(
                dst2_hbm.at[pl.ds(row0 + block * KI, KI)],
                idxd.at[pl.ds(slot * KI, KI)], isem)

        idx_cp(0, 0).start()

        @pl.loop(0, NB, step=2)
        def _blocks(kv):
            for p in (0, 1):
                k = kv + p
                idx_cp(k, p).wait()

                @pl.when(k + 1 < NB)
                def _():
                    idx_cp(k + 1, p ^ 1).start()

                for hb in range(2):
                    sc = [pltpu.make_async_copy(
                        ones_v, acc.at[idxd.at[p * KI + hb * (KI // 2) + r]],
                        ssem) for r in range(KI // 2)]
                    for c in sc:
                        c.start(add=True)
                    for c in sc:
                        c.wait()

        plsc.subcore_barrier()
        pltpu.sync_copy(acc.at[pl.ds(sid * RPT, RPT)], stage)
        pltpu.sync_copy(stage, out_hbm.at[pl.ds(cid * Np + sid * RPT, RPT)])

    return degk


def _make_prop_kernel(Np, E_pad):
    EP = E_pad // NS
    RPW = EP // CH
    NB = RPW // KI
    RPT = Np // NS
    ZB = RPT // 8
    mesh = plsc.VectorSubcoreMesh(core_axis_name="c", subcore_axis_name="s",
                                  num_cores=NC)

    @functools.partial(
        pl.kernel,
        mesh=mesh,
        compiler_params=pltpu.CompilerParams(use_tc_tiling_on_sc=False),
        out_type=jax.ShapeDtypeStruct((NC * Np, FH), jnp.float32),
        scratch_types=[
            pltpu.VMEM((2 * KI, CH), jnp.int32),
            pltpu.VMEM((2 * KI, CH), jnp.int32),
            pltpu.VMEM((2 * G, CH, FH), jnp.float32),
            pltpu.VMEM((ZB, FH), jnp.float32),
            pltpu.VMEM_SHARED((Np, FH), jnp.float32),
            pltpu.SemaphoreType.DMA,
            pltpu.SemaphoreType.DMA,
            pltpu.SemaphoreType.DMA,
        ],
    )
    def prop(src2_hbm, dst2_hbm, tlo_hbm, thi_hbm, out_hbm,
             idxs, idxd, rows, stage, acc, isem, gsem, ssem):
        cid = lax.axis_index("c")
        sid = lax.axis_index("s")

        def zfill(j, carry):
            stage[j, :] = jnp.zeros((FH,), jnp.float32)
            return carry
        lax.fori_loop(0, ZB, zfill, 0)

        def zcopy(k, carry):
            pltpu.sync_copy(stage, acc.at[pl.ds(sid * RPT + k * ZB, ZB)])
            return carry
        lax.fori_loop(0, 8, zcopy, 0)
        plsc.subcore_barrier()

        row0 = sid * RPW

        def idx_cp(ref, buf, block, slot):
            return pltpu.make_async_copy(
                ref.at[pl.ds(row0 + block * KI, KI)],
                buf.at[pl.ds(slot * KI, KI)], isem)

        def gat_cp(table, p, h, i):
            return pltpu.make_async_copy(
                table.at[idxs.at[p * KI + h * G + i]],
                rows.at[(h % 2) * G + i], gsem)

        def sca_cp(p, h, i):
            return pltpu.make_async_copy(
                rows.at[(h % 2) * G + i],
                acc.at[idxd.at[p * KI + h * G + i]], ssem)

        def run_edges(table):
            idx_cp(src2_hbm, idxs, 0, 0).start()
            idx_cp(dst2_hbm, idxd, 0, 0).start()

            @pl.loop(0, NB, step=2)
            def _blocks(kv):
                for p in (0, 1):
                    k = kv + p
                    idx_cp(src2_hbm, idxs, k, p).wait()
                    idx_cp(dst2_hbm, idxd, k, p).wait()

                    @pl.when(k + 1 < NB)
                    def _():
                        idx_cp(src2_hbm, idxs, k + 1, p ^ 1).start()
                        idx_cp(dst2_hbm, idxd, k + 1, p ^ 1).start()

                    for h in range(KI // G):
                        gs = [gat_cp(table, p, h, i) for i in range(G)]
                        for c in gs:
                            c.start()
                        for c in gs:
                            c.wait()
                        if h > 0:
                            for i in range(G):
                                sca_cp(p, h - 1, i).wait()
                        else:
                            @pl.when(k > 0)
                            def _():
                                for i in range(G):
                                    sca_cp(p ^ 1, KI // G - 1, i).wait()
                        for i in range(G):
                            sca_cp(p, h, i).start(add=True)

            for i in range(G):
                sca_cp((NB - 1) % 2, KI // G - 1, i).wait()

        @pl.when(cid == 0)
        def _():
            run_edges(tlo_hbm)

        @pl.when(cid == 1)
        def _():
            run_edges(thi_hbm)

        plsc.subcore_barrier()

        def ocopy(k, carry):
            pltpu.sync_copy(acc.at[pl.ds(sid * RPT + k * ZB, ZB)], stage)
            pltpu.sync_copy(
                stage, out_hbm.at[pl.ds(cid * Np + sid * RPT + k * ZB, ZB)])
            return carry
        lax.fori_loop(0, 8, ocopy, 0)

    return prop


def _row_grid(Np):
    return (pl.cdiv(Np, BLK),)


def _bspecF():
    return pl.BlockSpec((BLK, F), lambda i: (i, 0))


def _bspecH():
    return pl.BlockSpec((BLK, FH), lambda i: (i, 0))


def _bspec1():
    return pl.BlockSpec((BLK, 1), lambda i: (i, 0))


def _bspecW():
    return pl.BlockSpec((F, F), lambda i: (0, 0))


def _bspecB():
    return pl.BlockSpec((1, F), lambda i: (0, 0))


def _dis_body(d0, d1, o):
    o[...] = lax.rsqrt(d0[...] + d1[...] + 1.0)


def _make_dis_kernel(Np):
    return pl.pallas_call(
        _dis_body,
        grid=_row_grid(Np),
        in_specs=[_bspec1(), _bspec1()],
        out_specs=_bspec1(),
        out_shape=jax.ShapeDtypeStruct((Np, 1), jnp.float32),
    )


def _in_body(x, dis, w, olo, ohi):
    r = dis[...] * jnp.dot(x[...], w[...], preferred_element_type=jnp.float32)
    olo[...] = r[:, :FH]
    ohi[...] = r[:, FH:]


def _make_in_kernel(Np):
    return pl.pallas_call(
        _in_body,
        grid=_row_grid(Np),
        in_specs=[_bspecF(), _bspec1(), _bspecW()],
        out_specs=[_bspecH(), _bspecH()],
        out_shape=[jax.ShapeDtypeStruct((Np, FH), jnp.float32),
                   jax.ShapeDtypeStruct((Np, FH), jnp.float32)],
    )


def _mid_body(plo, phi, xlo, xhi, dis, b, w, olo, ohi):
    d = dis[...]
    p = jnp.concatenate([plo[...], phi[...]], axis=1)
    xt = jnp.concatenate([xlo[...], xhi[...]], axis=1)
    z = d * (p + xt) + b[...]
    h = 0.5 * z * (1.0 + lax.erf(z * _RS2))
    r = d * jnp.dot(h, w[...], preferred_element_type=jnp.float32)
    olo[...] = r[:, :FH]
    ohi[...] = r[:, FH:]


def _make_mid_kernel(Np):
    return pl.pallas_call(
        _mid_body,
        grid=_row_grid(Np),
        in_specs=[_bspecH(), _bspecH(), _bspecH(), _bspecH(),
                  _bspec1(), _bspecB(), _bspecW()],
        out_specs=[_bspecH(), _bspecH()],
        out_shape=[jax.ShapeDtypeStruct((Np, FH), jnp.float32),
                   jax.ShapeDtypeStruct((Np, FH), jnp.float32)],
    )


def _make_out_kernel(Np, OUT):
    def body(plo, phi, xlo, xhi, dis, b, o):
        p = jnp.concatenate([plo[...], phi[...]], axis=1)
        xt = jnp.concatenate([xlo[...], xhi[...]], axis=1)
        z = dis[...] * (p + xt) + b[...]
        col = lax.broadcasted_iota(jnp.int32, (BLK, F), 1)
        valid = col < OUT
        zm = jnp.where(valid, z, -jnp.inf)
        m = jnp.max(zm, axis=1, keepdims=True)
        e = jnp.where(valid, jnp.exp(z - m), 0.0)
        s = jnp.sum(e, axis=1, keepdims=True)
        o[...] = (z - m - jnp.log(s))[:, :OUT]

    return pl.pallas_call(
        body,
        grid=_row_grid(Np),
        in_specs=[_bspecH(), _bspecH(), _bspecH(), _bspecH(),
                  _bspec1(), _bspecB()],
        out_specs=pl.BlockSpec((BLK, OUT), lambda i: (i, 0)),
        out_shape=jax.ShapeDtypeStruct((Np, OUT), jnp.float32),
    )


def kernel(X, A, W1, b1, W2, b2, W3, b3):
    N, K = X.shape
    E = A.shape[1]
    OUT = W3.shape[1]
    Np = _round_up(N, 256) + 256
    E_pad = _round_up(E, NC * NS * CH * KI)

    src = A[0].astype(jnp.int32)
    dst = A[1].astype(jnp.int32)
    npad = E_pad - E
    pad_node = N + (jnp.arange(npad, dtype=jnp.int32) % 128)
    srcp = jnp.concatenate([src, pad_node]).reshape(E_pad // CH, CH)
    dstp = jnp.concatenate([dst, pad_node]).reshape(E_pad // CH, CH)

    X_pad = jnp.pad(X, ((0, Np - N), (0, F - K)))
    W1p = jnp.pad(W1, ((0, F - K), (0, F - K)))
    W2p = jnp.pad(W2, ((0, F - K), (0, F - K)))
    W3p = jnp.pad(W3, ((0, F - K), (0, F - OUT)))
    b1p = jnp.pad(b1, (0, F - K)).reshape(1, F)
    b2p = jnp.pad(b2, (0, F - K)).reshape(1, F)
    b3p = jnp.pad(b3, (0, F - OUT)).reshape(1, F)

    degk = _make_deg_kernel(Np, E_pad)
    prop = _make_prop_kernel(Np, E_pad)
    k_dis = _make_dis_kernel(Np)
    k_in = _make_in_kernel(Np)
    k_mid = _make_mid_kernel(Np)
    k_out = _make_out_kernel(Np, OUT)

    degp = degk(dstp)
    dis = k_dis(degp[:Np].reshape(Np, 1), degp[Np:].reshape(Np, 1))

    lo, hi = k_in(X_pad, dis, W1p)
    p = prop(srcp, dstp, lo, hi)
    lo, hi = k_mid(p[:Np], p[Np:], lo, hi, dis, b1p, W2p)
    p = prop(srcp, dstp, lo, hi)
    lo, hi = k_mid(p[:Np], p[Np:], lo, hi, dis, b2p, W3p)
    p = prop(srcp, dstp, lo, hi)
    out = k_out(p[:Np], p[Np:], lo, hi, dis, b3p)
    return out[:N]


# speedup vs baseline: 48.2393x; 1.0973x over previous
import functools

import jax
import jax.numpy as jnp
from jax import lax
from jax.experimental import pallas as pl
from jax.experimental.pallas import tpu as pltpu
from jax.experimental.pallas import tpu_sc as plsc

F = 16
FH = 8
NC = 2
NS = 16
CH = 128
G = 4
KI = 16
BLK = 1024
_RS2 = 0.7071067811865476


def _round_up(x, m):
    return (x + m - 1) // m * m


def _make_deg_kernel(Np, E_pad):
    W = NC * NS
    EP = E_pad // W
    RPW = EP // CH
    NB = RPW // KI
    RPT = Np // NS
    mesh = plsc.VectorSubcoreMesh(core_axis_name="c", subcore_axis_name="s",
                                  num_cores=NC)

    @functools.partial(
        pl.kernel,
        mesh=mesh,
        compiler_params=pltpu.CompilerParams(use_tc_tiling_on_sc=False),
        out_type=jax.ShapeDtypeStruct((NC * Np,), jnp.float32),
        scratch_types=[
            pltpu.VMEM((2 * KI, CH), jnp.int32),
            pltpu.VMEM((CH,), jnp.float32),
            pltpu.VMEM((RPT,), jnp.float32),
            pltpu.VMEM_SHARED((Np,), jnp.float32),
            pltpu.SemaphoreType.DMA,
            pltpu.SemaphoreType.DMA,
        ],
    )
    def degk(dst2_hbm, out_hbm, idxd, ones_v, stage, acc, isem, ssem):
        cid = lax.axis_index("c")
        sid = lax.axis_index("s")

        def zfill(j, carry):
            stage[pl.ds(j * 16, 16)] = jnp.zeros((16,), jnp.float32)
            return carry
        lax.fori_loop(0, RPT // 16, zfill, 0)
        pltpu.sync_copy(stage, acc.at[pl.ds(sid * RPT, RPT)])
        for i in range(CH // 16):
            ones_v[pl.ds(i * 16, 16)] = jnp.full((16,), 1.0, jnp.float32)
        plsc.subcore_barrier()

        row0 = (cid * NS + sid) * RPW

        def idx_cp(block, slot):
            return pltpu.make_async_copy(
                dst2_hbm.at[pl.ds(row0 + block * KI, KI)],
                idxd.at[pl.ds(slot * KI, KI)], isem)

        idx_cp(0, 0).start()

        @pl.loop(0, NB, step=2)
        def _blocks(kv):
            for p in (0, 1):
                k = kv + p
                idx_cp(k, p).wait()

                @pl.when(k + 1 < NB)
                def _():
                    idx_cp(k + 1, p ^ 1).start()

                for hb in range(2):
                    sc = [pltpu.make_async_copy(
                        ones_v, acc.at[idxd.at[p * KI + hb * (KI // 2) + r]],
                        ssem) for r in range(KI // 2)]
                    for c in sc:
                        c.start(add=True)
                    for c in sc:
                        c.wait()

        plsc.subcore_barrier()
        pltpu.sync_copy(acc.at[pl.ds(sid * RPT, RPT)], stage)
        pltpu.sync_copy(stage, out_hbm.at[pl.ds(cid * Np + sid * RPT, RPT)])

    return degk


def _make_prop_kernel(Np, E_pad):
    EP = E_pad // NS
    RPW = EP // CH
    NB = RPW // KI
    RPT = Np // NS
    ZB = RPT // 8
    mesh = plsc.VectorSubcoreMesh(core_axis_name="c", subcore_axis_name="s",
                                  num_cores=NC)

    @functools.partial(
        pl.kernel,
        mesh=mesh,
        compiler_params=pltpu.CompilerParams(use_tc_tiling_on_sc=False),
        out_type=jax.ShapeDtypeStruct((NC * Np, FH), jnp.float32),
        scratch_types=[
            pltpu.VMEM((2 * KI, CH), jnp.int32),
            pltpu.VMEM((2 * KI, CH), jnp.int32),
            pltpu.VMEM((4 * G, CH, FH), jnp.float32),
            pltpu.VMEM((ZB, FH), jnp.float32),
            pltpu.VMEM_SHARED((Np, FH), jnp.float32),
            pltpu.SemaphoreType.DMA,
            pltpu.SemaphoreType.DMA,
            pltpu.SemaphoreType.DMA,
        ],
    )
    def prop(src2_hbm, dst2_hbm, tlo_hbm, thi_hbm, out_hbm,
             idxs, idxd, rows, stage, acc, isem, gsem, ssem):
        cid = lax.axis_index("c")
        sid = lax.axis_index("s")

        def zfill(j, carry):
            stage[j, :] = jnp.zeros((FH,), jnp.float32)
            return carry
        lax.fori_loop(0, ZB, zfill, 0)

        def zcopy(k, carry):
            pltpu.sync_copy(stage, acc.at[pl.ds(sid * RPT + k * ZB, ZB)])
            return carry
        lax.fori_loop(0, 8, zcopy, 0)
        plsc.subcore_barrier()

        row0 = sid * RPW

        def idx_cp(ref, buf, block, slot):
            return pltpu.make_async_copy(
                ref.at[pl.ds(row0 + block * KI, KI)],
                buf.at[pl.ds(slot * KI, KI)], isem)

        def gat_cp(table, p, h, i):
            return pltpu.make_async_copy(
                table.at[idxs.at[p * KI + h * G + i]],
                rows.at[h * G + i], gsem)

        def sca_cp(p, h, i):
            return pltpu.make_async_copy(
                rows.at[h * G + i],
                acc.at[idxd.at[p * KI + h * G + i]], ssem)

        NSB = KI // G

        def run_edges(table):
            idx_cp(src2_hbm, idxs, 0, 0).start()
            idx_cp(dst2_hbm, idxd, 0, 0).start()
            idx_cp(src2_hbm, idxs, 0, 0).wait()
            idx_cp(dst2_hbm, idxd, 0, 0).wait()
            for i in range(G):
                gat_cp(table, 0, 0, i).start()
            for i in range(G):
                gat_cp(table, 0, 1, i).start()

            @pl.loop(0, NB, step=2)
            def _blocks(kv):
                for p in (0, 1):
                    k = kv + p
                    for h in range(NSB):
                        for i in range(G):
                            gat_cp(table, p, h, i).wait()
                        if h > 0:
                            for i in range(G):
                                sca_cp(p, h - 1, i).wait()
                        else:
                            @pl.when(k > 0)
                            def _():
                                for i in range(G):
                                    sca_cp(p ^ 1, NSB - 1, i).wait()
                            @pl.when(k + 1 < NB)
                            def _():
                                idx_cp(src2_hbm, idxs, k + 1, p ^ 1).start()
                                idx_cp(dst2_hbm, idxd, k + 1, p ^ 1).start()
                        if h + 2 < NSB:
                            for i in range(G):
                                gat_cp(table, p, h + 2, i).start()
                        else:
                            nh = h + 2 - NSB
                            if nh == 0:
                                @pl.when(k + 1 < NB)
                                def _():
                                    idx_cp(src2_hbm, idxs,
                                           k + 1, p ^ 1).wait()
                                    idx_cp(dst2_hbm, idxd,
                                           k + 1, p ^ 1).wait()
                                    for i in range(G):
                                        gat_cp(table, p ^ 1, 0, i).start()
                            else:
                                @pl.when(k + 1 < NB)
                                def _():
                                    for i in range(G):
                                        gat_cp(table, p ^ 1, nh, i).start()
                        for i in range(G):
                            sca_cp(p, h, i).start(add=True)

            for i in range(G):
                sca_cp((NB - 1) % 2, NSB - 1, i).wait()

        @pl.when(cid == 0)
        def _():
            run_edges(tlo_hbm)

        @pl.when(cid == 1)
        def _():
            run_edges(thi_hbm)

        plsc.subcore_barrier()

        def ocopy(k, carry):
            pltpu.sync_copy(acc.at[pl.ds(sid * RPT + k * ZB, ZB)], stage)
            pltpu.sync_copy(
                stage, out_hbm.at[pl.ds(cid * Np + sid * RPT + k * ZB, ZB)])
            return carry
        lax.fori_loop(0, 8, ocopy, 0)

    return prop


def _row_grid(Np):
    return (pl.cdiv(Np, BLK),)


def _bspecF():
    return pl.BlockSpec((BLK, F), lambda i: (i, 0))


def _bspecH():
    return pl.BlockSpec((BLK, FH), lambda i: (i, 0))


def _bspec1():
    return pl.BlockSpec((BLK, 1), lambda i: (i, 0))


def _bspecW():
    return pl.BlockSpec((F, F), lambda i: (0, 0))


def _bspecB():
    return pl.BlockSpec((1, F), lambda i: (0, 0))


def _dis_body(d0, d1, o):
    o[...] = lax.rsqrt(d0[...] + d1[...] + 1.0)


def _make_dis_kernel(Np):
    return pl.pallas_call(
        _dis_body,
        grid=_row_grid(Np),
        in_specs=[_bspec1(), _bspec1()],
        out_specs=_bspec1(),
        out_shape=jax.ShapeDtypeStruct((Np, 1), jnp.float32),
    )


def _in_body(x, dis, w, olo, ohi):
    r = dis[...] * jnp.dot(x[...], w[...], preferred_element_type=jnp.float32)
    olo[...] = r[:, :FH]
    ohi[...] = r[:, FH:]


def _make_in_kernel(Np):
    return pl.pallas_call(
        _in_body,
        grid=_row_grid(Np),
        in_specs=[_bspecF(), _bspec1(), _bspecW()],
        out_specs=[_bspecH(), _bspecH()],
        out_shape=[jax.ShapeDtypeStruct((Np, FH), jnp.float32),
                   jax.ShapeDtypeStruct((Np, FH), jnp.float32)],
    )


def _mid_body(plo, phi, xlo, xhi, dis, b, w, olo, ohi):
    d = dis[...]
    p = jnp.concatenate([plo[...], phi[...]], axis=1)
    xt = jnp.concatenate([xlo[...], xhi[...]], axis=1)
    z = d * (p + xt) + b[...]
    h = 0.5 * z * (1.0 + lax.erf(z * _RS2))
    r = d * jnp.dot(h, w[...], preferred_element_type=jnp.float32)
    olo[...] = r[:, :FH]
    ohi[...] = r[:, FH:]


def _make_mid_kernel(Np):
    return pl.pallas_call(
        _mid_body,
        grid=_row_grid(Np),
        in_specs=[_bspecH(), _bspecH(), _bspecH(), _bspecH(),
                  _bspec1(), _bspecB(), _bspecW()],
        out_specs=[_bspecH(), _bspecH()],
        out_shape=[jax.ShapeDtypeStruct((Np, FH), jnp.float32),
                   jax.ShapeDtypeStruct((Np, FH), jnp.float32)],
    )


def _make_out_kernel(Np, OUT):
    def body(plo, phi, xlo, xhi, dis, b, o):
        p = jnp.concatenate([plo[...], phi[...]], axis=1)
        xt = jnp.concatenate([xlo[...], xhi[...]], axis=1)
        z = dis[...] * (p + xt) + b[...]
        col = lax.broadcasted_iota(jnp.int32, (BLK, F), 1)
        valid = col < OUT
        zm = jnp.where(valid, z, -jnp.inf)
        m = jnp.max(zm, axis=1, keepdims=True)
        e = jnp.where(valid, jnp.exp(z - m), 0.0)
        s = jnp.sum(e, axis=1, keepdims=True)
        o[...] = (z - m - jnp.log(s))[:, :OUT]

    return pl.pallas_call(
        body,
        grid=_row_grid(Np),
        in_specs=[_bspecH(), _bspecH(), _bspecH(), _bspecH(),
                  _bspec1(), _bspecB()],
        out_specs=pl.BlockSpec((BLK, OUT), lambda i: (i, 0)),
        out_shape=jax.ShapeDtypeStruct((Np, OUT), jnp.float32),
    )


def kernel(X, A, W1, b1, W2, b2, W3, b3):
    N, K = X.shape
    E = A.shape[1]
    OUT = W3.shape[1]
    Np = _round_up(N, 256) + 256
    E_pad = _round_up(E, NC * NS * CH * KI)

    src = A[0].astype(jnp.int32)
    dst = A[1].astype(jnp.int32)
    npad = E_pad - E
    pad_node = N + (jnp.arange(npad, dtype=jnp.int32) % 128)
    srcp = jnp.concatenate([src, pad_node]).reshape(E_pad // CH, CH)
    dstp = jnp.concatenate([dst, pad_node]).reshape(E_pad // CH, CH)

    X_pad = jnp.pad(X, ((0, Np - N), (0, F - K)))
    W1p = jnp.pad(W1, ((0, F - K), (0, F - K)))
    W2p = jnp.pad(W2, ((0, F - K), (0, F - K)))
    W3p = jnp.pad(W3, ((0, F - K), (0, F - OUT)))
    b1p = jnp.pad(b1, (0, F - K)).reshape(1, F)
    b2p = jnp.pad(b2, (0, F - K)).reshape(1, F)
    b3p = jnp.pad(b3, (0, F - OUT)).reshape(1, F)

    degk = _make_deg_kernel(Np, E_pad)
    prop = _make_prop_kernel(Np, E_pad)
    k_dis = _make_dis_kernel(Np)
    k_in = _make_in_kernel(Np)
    k_mid = _make_mid_kernel(Np)
    k_out = _make_out_kernel(Np, OUT)

    degp = degk(dstp)
    dis = k_dis(degp[:Np].reshape(Np, 1), degp[Np:].reshape(Np, 1))

    lo, hi = k_in(X_pad, dis, W1p)
    p = prop(srcp, dstp, lo, hi)
    lo, hi = k_mid(p[:Np], p[Np:], lo, hi, dis, b1p, W2p)
    p = prop(srcp, dstp, lo, hi)
    lo, hi = k_mid(p[:Np], p[Np:], lo, hi, dis, b2p, W3p)
    p = prop(srcp, dstp, lo, hi)
    out = k_out(p[:Np], p[Np:], lo, hi, dis, b3p)
    return out[:N]
